# Initial kernel scaffold; baseline (speedup 1.0000x reference)
import jax
import jax.numpy as jnp
from jax.experimental import pallas as pl


def kernel(state, W_embed, b_embed, W1, b1, W2, b2, W_dec, b_dec):
    raise NotImplementedError("write your pallas kernel here")


# speedup vs baseline: 1027.0006x; 1027.0006x over previous
import jax
import jax.numpy as jnp
from jax.experimental import pallas as pl

S = 5000
W = 100
F = 8
D = 32


def _policy_body(state_ref, we_ref, be_ref, w1_ref, b1_ref, w2_ref, b2_ref,
                 wd_ref, bd_ref, out_ref):
    feats = state_ref[:, 0:F]
    mean_r = jnp.sum(feats, axis=0, keepdims=True) * (1.0 / S)

    mw = jnp.dot(mean_r, we_ref[...],
                 preferred_element_type=jnp.float32) + be_ref[...]

    h1w = jax.nn.relu(jnp.dot(mw, w1_ref[...],
                              preferred_element_type=jnp.float32) + b1_ref[...])
    h1s = jax.nn.relu(b1_ref[...])
    h2s = jnp.dot(h1w, w2_ref[...],
                  preferred_element_type=jnp.float32) + b2_ref[...]
    h2w = jnp.dot(h1s, w2_ref[...],
                  preferred_element_type=jnp.float32) + b2_ref[...]

    dec_in = jnp.concatenate([h2s, h2w], axis=1)
    score = jnp.dot(dec_in, wd_ref[...],
                    preferred_element_type=jnp.float32) + bd_ref[...]
    srow = jnp.broadcast_to(score, (1, W))
    m = jnp.max(srow, axis=1, keepdims=True)
    e = jnp.exp(srow - m)
    out_ref[...] = e / jnp.sum(e, axis=1, keepdims=True)


def kernel(state, W_embed, b_embed, W1, b1, W2, b2, W_dec, b_dec):
    out = pl.pallas_call(
        _policy_body,
        out_shape=jax.ShapeDtypeStruct((1, W), jnp.float32),
    )(state,
      W_embed, b_embed.reshape(1, D),
      W1, b1.reshape(1, D),
      W2, b2.reshape(1, D),
      W_dec, b_dec.reshape(1, 1))
    return out.reshape(W)
